# baseline (speedup 1.0000x reference)
import jax
import jax.numpy as jnp
from jax import lax
from jax.experimental import pallas as pl
from jax.experimental.pallas import tpu as pltpu
from jax.experimental.pallas import tpu_sc as plsc

B, C, H, W = 32, 256, 64, 64
HW = H * W
N = 20000
L = 16
NW = 32
NCHUNK = N // L
CPW = (NCHUNK + NW - 1) // NW
NB = C // L

_DNUMS = lax.GatherDimensionNumbers(
    offset_dims=(), collapsed_slice_dims=(0,), start_index_map=(0,))


def _lane_gather(vec, idx):
    return lax.gather(vec, idx[:, None], _DNUMS, (1,),
                      mode=lax.GatherScatterMode.PROMISE_IN_BOUNDS)


def _floor_i32(x):
    t = x.astype(jnp.int32)
    return jnp.where(t.astype(jnp.float32) > x, t - 1, t)


def _tc_maxpool_body(t_ref, mp_ref):
    x = t_ref[...]
    r = lax.broadcasted_iota(jnp.int32, (HW, C), 0)
    xl = jnp.concatenate([x[:1], x[:-1]], axis=0)
    xl = jnp.where(jnp.bitwise_and(r, W - 1) == 0, x, xl)
    xr = jnp.concatenate([x[1:], x[-1:]], axis=0)
    xr = jnp.where(jnp.bitwise_and(r, W - 1) == W - 1, x, xr)
    m = jnp.maximum(jnp.maximum(xl, x), xr)
    yu = jnp.concatenate([m[:W], m[:-W]], axis=0)
    yu = jnp.where(r < W, m, yu)
    yd = jnp.concatenate([m[W:], m[-W:]], axis=0)
    yd = jnp.where(r >= HW - W, m, yd)
    mp_ref[...] = jnp.maximum(jnp.maximum(yu, m), yd)


def _sc_body(tab, mpt, dyt, dxt, bids, inds, out,
             bid_v, ind_v, dy_v, dx_v, w_v, idx3_v,
             cen_v, mp_v, cor_v, o1_v,
             sem_i, sem_o, sem_b0, sem_b1, sem_c0, sem_c1, sem_w):
    wid = lax.axis_index("s") * 2 + lax.axis_index("c")
    sem_b = (sem_b0, sem_b1)
    sem_c = (sem_c0, sem_c1)

    def fire_chunk(q):
        bid = bid_v[q]
        ind = ind_v[q]
        rows_c = bid * HW + ind
        pltpu.async_copy(dyt.at[rows_c], dy_v.at[q], sem_o)
        pltpu.async_copy(dxt.at[rows_c], dx_v.at[q], sem_o)
        pltpu.async_copy(tab.at[rows_c], cen_v.at[q], sem_b[q])
        pltpu.async_copy(mpt.at[rows_c], mp_v.at[q], sem_b[q])

    def corner_stage(q):
        pltpu.make_async_copy(
            dyt.at[pl.ds(0, L)], dy_v.at[q], sem_o).wait()
        pltpu.make_async_copy(
            dxt.at[pl.ds(0, L)], dx_v.at[q], sem_o).wait()
        ind = ind_v[q]
        brow = bid_v[q] * HW
        cy = lax.shift_right_logical(ind, 6)
        cx = jnp.bitwise_and(ind, 63)
        y = cy.astype(jnp.float32) + dy_v[q]
        x = cx.astype(jnp.float32) + dx_v[q]
        x0 = _floor_i32(x)
        y0 = _floor_i32(y)
        x1 = x0 + 1
        y1 = y0 + 1
        wx1 = x - x0.astype(jnp.float32)
        wx0 = 1.0 - wx1
        wy1 = y - y0.astype(jnp.float32)
        wy0 = 1.0 - wy1
        corners = ((y0, x0, wy0 * wx0), (y0, x1, wy0 * wx1),
                   (y1, x0, wy1 * wx0), (y1, x1, wy1 * wx1))
        for k, (yi, xi, wk) in enumerate(corners):
            valid = ((xi >= 0) & (xi <= W - 1)
                     & (yi >= 0) & (yi <= H - 1))
            xc = jnp.minimum(jnp.maximum(xi, 0), W - 1)
            yc = jnp.minimum(jnp.maximum(yi, 0), H - 1)
            idx3_v[q, pl.ds(k * L, L)] = brow + yc * W + xc
            w_v[q, k] = jnp.where(valid, wk, 0.0)
        pltpu.async_copy(tab.at[idx3_v.at[q]], cor_v.at[q], sem_c[q])

    def half(t, p):
        ci = wid + NW * t
        nxt = ci + NW
        q = 1 - p

        @pl.when(ci < NCHUNK)
        def _():
            base = pl.multiple_of(ci * L, L)

            @pl.when(t > 0)
            def _():
                pltpu.make_async_copy(
                    cen_v.at[p], out.at[0, pl.ds(base, L)], sem_w).wait()
                pltpu.make_async_copy(
                    o1_v.at[p], out.at[1, pl.ds(base, L)], sem_w).wait()
                pltpu.make_async_copy(
                    mp_v.at[p], out.at[2, pl.ds(base, L)], sem_w).wait()

            @pl.when(nxt < NCHUNK)
            def _():
                pltpu.make_async_copy(
                    bids.at[pl.ds(0, L)], bid_v.at[q], sem_i).wait()
                pltpu.make_async_copy(
                    inds.at[pl.ds(0, L)], ind_v.at[q], sem_i).wait()
                fire_chunk(q)

            @pl.when(nxt + NW < NCHUNK)
            def _():
                nbase = pl.multiple_of((nxt + NW) * L, L)
                pltpu.async_copy(bids.at[pl.ds(nbase, L)],
                                 bid_v.at[p], sem_i)
                pltpu.async_copy(inds.at[pl.ds(nbase, L)],
                                 ind_v.at[p], sem_i)

            pltpu.make_async_copy(
                tab.at[pl.ds(0, L)], cen_v.at[p], sem_b[p]).wait()
            pltpu.make_async_copy(
                mpt.at[pl.ds(0, L)], mp_v.at[p], sem_b[p]).wait()
            pltpu.async_copy(cen_v.at[p], out.at[0, pl.ds(base, L)], sem_w)
            pltpu.async_copy(mp_v.at[p], out.at[2, pl.ds(base, L)], sem_w)

            pltpu.make_async_copy(
                tab.at[pl.ds(0, 4 * L)], cor_v.at[p], sem_c[p]).wait()

            def bil_body(d, _):
                d_idx = jnp.broadcast_to(d, (L,))
                wb = [_lane_gather(w_v[p, k], d_idx) for k in range(4)]
                for j in range(NB):
                    sl = pl.ds(j * L, L)
                    acc = wb[0] * cor_v[p, d, sl]
                    acc = acc + wb[1] * cor_v[p, L + d, sl]
                    acc = acc + wb[2] * cor_v[p, 2 * L + d, sl]
                    acc = acc + wb[3] * cor_v[p, 3 * L + d, sl]
                    o1_v[p, d, sl] = acc
                return _

            lax.fori_loop(0, L, bil_body, None)
            pltpu.async_copy(o1_v.at[p], out.at[1, pl.ds(base, L)], sem_w)

            @pl.when(nxt < NCHUNK)
            def _():
                corner_stage(q)

    base0 = pl.multiple_of(wid * L, L)
    pltpu.async_copy(bids.at[pl.ds(base0, L)], bid_v.at[0], sem_i).wait()
    pltpu.async_copy(inds.at[pl.ds(base0, L)], ind_v.at[0], sem_i).wait()
    fire_chunk(0)

    @pl.when(wid + NW < NCHUNK)
    def _():
        base1 = pl.multiple_of((wid + NW) * L, L)
        pltpu.async_copy(bids.at[pl.ds(base1, L)], bid_v.at[1], sem_i)
        pltpu.async_copy(inds.at[pl.ds(base1, L)], ind_v.at[1], sem_i)

    corner_stage(0)

    def pair_body(u, _):
        half(2 * u, 0)
        half(2 * u + 1, 1)
        return _

    lax.fori_loop(0, CPW // 2, pair_body, None)

    pltpu.make_async_copy(cen_v.at[0], out.at[0, pl.ds(0, L)], sem_w).wait()
    pltpu.make_async_copy(o1_v.at[0], out.at[1, pl.ds(0, L)], sem_w).wait()
    pltpu.make_async_copy(mp_v.at[0], out.at[2, pl.ds(0, L)], sem_w).wait()


@jax.jit
def kernel(params_maps, offset_maps, batch_ids, flat_inds):
    tab = jnp.transpose(params_maps, (0, 2, 3, 1)).reshape(B * HW, C)
    mpt = pl.pallas_call(
        _tc_maxpool_body,
        grid=(B,),
        in_specs=[pl.BlockSpec((HW, C), lambda b: (b, 0))],
        out_specs=pl.BlockSpec((HW, C), lambda b: (b, 0)),
        out_shape=jax.ShapeDtypeStruct((B * HW, C), jnp.float32),
    )(tab)
    dyt = offset_maps[:, 0, :, :].reshape(B * HW)
    dxt = offset_maps[:, 1, :, :].reshape(B * HW)

    mesh = plsc.VectorSubcoreMesh(core_axis_name="c", subcore_axis_name="s")
    f = pl.kernel(
        _sc_body,
        mesh=mesh,
        out_type=jax.ShapeDtypeStruct((3, N, C), jnp.float32),
        scratch_types=[
            pltpu.VMEM((2, L), jnp.int32),
            pltpu.VMEM((2, L), jnp.int32),
            pltpu.VMEM((2, L), jnp.float32),
            pltpu.VMEM((2, L), jnp.float32),
            pltpu.VMEM((2, 4, L), jnp.float32),
            pltpu.VMEM((2, 4 * L), jnp.int32),
            pltpu.VMEM((2, L, C), jnp.float32),
            pltpu.VMEM((2, L, C), jnp.float32),
            pltpu.VMEM((2, 4 * L, C), jnp.float32),
            pltpu.VMEM((2, L, C), jnp.float32),
            pltpu.SemaphoreType.DMA,
            pltpu.SemaphoreType.DMA,
            pltpu.SemaphoreType.DMA,
            pltpu.SemaphoreType.DMA,
            pltpu.SemaphoreType.DMA,
            pltpu.SemaphoreType.DMA,
            pltpu.SemaphoreType.DMA,
        ],
    )
    return f(tab, mpt, dyt, dxt, batch_ids, flat_inds)


# speedup vs baseline: 1.1494x; 1.1494x over previous
import jax
import jax.numpy as jnp
from jax import lax
from jax.experimental import pallas as pl
from jax.experimental.pallas import tpu as pltpu
from jax.experimental.pallas import tpu_sc as plsc

B, C, H, W = 32, 256, 64, 64
HW = H * W
N = 20000
L = 16
NW = 32
NCHUNK = N // L
CPW = (NCHUNK + NW - 1) // NW
NB = C // L

_DNUMS = lax.GatherDimensionNumbers(
    offset_dims=(), collapsed_slice_dims=(0,), start_index_map=(0,))


def _lane_gather(vec, idx):
    return lax.gather(vec, idx[:, None], _DNUMS, (1,),
                      mode=lax.GatherScatterMode.PROMISE_IN_BOUNDS)


def _floor_i32(x):
    t = x.astype(jnp.int32)
    return jnp.where(t.astype(jnp.float32) > x, t - 1, t)


def _tc_maxpool_body(t_ref, mp_ref):
    x = t_ref[...]
    r = lax.broadcasted_iota(jnp.int32, (HW, C), 0)
    xl = jnp.concatenate([x[:1], x[:-1]], axis=0)
    xl = jnp.where(jnp.bitwise_and(r, W - 1) == 0, x, xl)
    xr = jnp.concatenate([x[1:], x[-1:]], axis=0)
    xr = jnp.where(jnp.bitwise_and(r, W - 1) == W - 1, x, xr)
    m = jnp.maximum(jnp.maximum(xl, x), xr)
    yu = jnp.concatenate([m[:W], m[:-W]], axis=0)
    yu = jnp.where(r < W, m, yu)
    yd = jnp.concatenate([m[W:], m[-W:]], axis=0)
    yd = jnp.where(r >= HW - W, m, yd)
    mp_ref[...] = jnp.maximum(jnp.maximum(yu, m), yd)


def _sc_body(tab, mpt, dyt, dxt, bids, inds, out,
             bid_v, ind_v, dy_v, dx_v, w_v, idx3_v,
             cen_v, mp_v, cor_v, o1_v,
             sem_i, sem_o, sem_b0, sem_b1, sem_c0, sem_c1, sem_w):
    wid = lax.axis_index("s") * 2 + lax.axis_index("c")
    sem_b = (sem_b0, sem_b1)
    sem_c = (sem_c0, sem_c1)

    def fire_chunk(q):
        bid = bid_v[q]
        ind = ind_v[q]
        rows_c = bid * HW + ind
        pltpu.async_copy(dyt.at[rows_c], dy_v.at[q], sem_o)
        pltpu.async_copy(dxt.at[rows_c], dx_v.at[q], sem_o)
        pltpu.async_copy(tab.at[rows_c], cen_v.at[q], sem_b[q])
        pltpu.async_copy(mpt.at[rows_c], mp_v.at[q], sem_b[q])

    def corner_stage(q):
        pltpu.make_async_copy(
            dyt.at[pl.ds(0, L)], dy_v.at[q], sem_o).wait()
        pltpu.make_async_copy(
            dxt.at[pl.ds(0, L)], dx_v.at[q], sem_o).wait()
        ind = ind_v[q]
        brow = bid_v[q] * HW
        cy = lax.shift_right_logical(ind, 6)
        cx = jnp.bitwise_and(ind, 63)
        y = cy.astype(jnp.float32) + dy_v[q]
        x = cx.astype(jnp.float32) + dx_v[q]
        x0 = _floor_i32(x)
        y0 = _floor_i32(y)
        x1 = x0 + 1
        y1 = y0 + 1
        wx1 = x - x0.astype(jnp.float32)
        wx0 = 1.0 - wx1
        wy1 = y - y0.astype(jnp.float32)
        wy0 = 1.0 - wy1
        corners = ((y0, x0, wy0 * wx0), (y0, x1, wy0 * wx1),
                   (y1, x0, wy1 * wx0), (y1, x1, wy1 * wx1))
        for k, (yi, xi, wk) in enumerate(corners):
            valid = ((xi >= 0) & (xi <= W - 1)
                     & (yi >= 0) & (yi <= H - 1))
            xc = jnp.minimum(jnp.maximum(xi, 0), W - 1)
            yc = jnp.minimum(jnp.maximum(yi, 0), H - 1)
            idx3_v[q, pl.ds(k * L, L)] = brow + yc * W + xc
            w_v[q, k] = jnp.where(valid, wk, 0.0)
        pltpu.async_copy(tab.at[idx3_v.at[q]], cor_v.at[q], sem_c[q])

    def half(t, p):
        ci = wid + NW * t
        nxt = ci + NW
        q = 1 - p

        @pl.when(ci < NCHUNK)
        def _():
            base = pl.multiple_of(ci * L, L)

            @pl.when(t > 0)
            def _():
                pltpu.make_async_copy(
                    cen_v.at[p], out.at[0, pl.ds(base, L)], sem_w).wait()
                pltpu.make_async_copy(
                    o1_v.at[p], out.at[1, pl.ds(base, L)], sem_w).wait()
                pltpu.make_async_copy(
                    mp_v.at[p], out.at[2, pl.ds(base, L)], sem_w).wait()

            @pl.when(nxt < NCHUNK)
            def _():
                pltpu.make_async_copy(
                    bids.at[pl.ds(0, L)], bid_v.at[q], sem_i).wait()
                pltpu.make_async_copy(
                    inds.at[pl.ds(0, L)], ind_v.at[q], sem_i).wait()
                fire_chunk(q)

            @pl.when(nxt + NW < NCHUNK)
            def _():
                nbase = pl.multiple_of((nxt + NW) * L, L)
                pltpu.async_copy(bids.at[pl.ds(nbase, L)],
                                 bid_v.at[p], sem_i)
                pltpu.async_copy(inds.at[pl.ds(nbase, L)],
                                 ind_v.at[p], sem_i)

            pltpu.make_async_copy(
                tab.at[pl.ds(0, L)], cen_v.at[p], sem_b[p]).wait()
            pltpu.make_async_copy(
                mpt.at[pl.ds(0, L)], mp_v.at[p], sem_b[p]).wait()
            pltpu.async_copy(cen_v.at[p], out.at[0, pl.ds(base, L)], sem_w)
            pltpu.async_copy(mp_v.at[p], out.at[2, pl.ds(base, L)], sem_w)

            @pl.when(nxt < NCHUNK)
            def _():
                corner_stage(q)

            pltpu.make_async_copy(
                tab.at[pl.ds(0, 4 * L)], cor_v.at[p], sem_c[p]).wait()

            def bil_body(d, _):
                d_idx = jnp.broadcast_to(d, (L,))
                wb = [_lane_gather(w_v[p, k], d_idx) for k in range(4)]
                for j in range(NB):
                    sl = pl.ds(j * L, L)
                    acc = wb[0] * cor_v[p, d, sl]
                    acc = acc + wb[1] * cor_v[p, L + d, sl]
                    acc = acc + wb[2] * cor_v[p, 2 * L + d, sl]
                    acc = acc + wb[3] * cor_v[p, 3 * L + d, sl]
                    o1_v[p, d, sl] = acc
                return _

            lax.fori_loop(0, L, bil_body, None)
            pltpu.async_copy(o1_v.at[p], out.at[1, pl.ds(base, L)], sem_w)

    base0 = pl.multiple_of(wid * L, L)
    pltpu.async_copy(bids.at[pl.ds(base0, L)], bid_v.at[0], sem_i).wait()
    pltpu.async_copy(inds.at[pl.ds(base0, L)], ind_v.at[0], sem_i).wait()
    fire_chunk(0)

    @pl.when(wid + NW < NCHUNK)
    def _():
        base1 = pl.multiple_of((wid + NW) * L, L)
        pltpu.async_copy(bids.at[pl.ds(base1, L)], bid_v.at[1], sem_i)
        pltpu.async_copy(inds.at[pl.ds(base1, L)], ind_v.at[1], sem_i)

    corner_stage(0)

    def pair_body(u, _):
        half(2 * u, 0)
        half(2 * u + 1, 1)
        return _

    lax.fori_loop(0, CPW // 2, pair_body, None)

    pltpu.make_async_copy(cen_v.at[0], out.at[0, pl.ds(0, L)], sem_w).wait()
    pltpu.make_async_copy(o1_v.at[0], out.at[1, pl.ds(0, L)], sem_w).wait()
    pltpu.make_async_copy(mp_v.at[0], out.at[2, pl.ds(0, L)], sem_w).wait()


@jax.jit
def kernel(params_maps, offset_maps, batch_ids, flat_inds):
    tab = jnp.transpose(params_maps, (0, 2, 3, 1)).reshape(B * HW, C)
    mpt = pl.pallas_call(
        _tc_maxpool_body,
        grid=(B,),
        in_specs=[pl.BlockSpec((HW, C), lambda b: (b, 0))],
        out_specs=pl.BlockSpec((HW, C), lambda b: (b, 0)),
        out_shape=jax.ShapeDtypeStruct((B * HW, C), jnp.float32),
    )(tab)
    dyt = offset_maps[:, 0, :, :].reshape(B * HW)
    dxt = offset_maps[:, 1, :, :].reshape(B * HW)

    mesh = plsc.VectorSubcoreMesh(core_axis_name="c", subcore_axis_name="s")
    f = pl.kernel(
        _sc_body,
        mesh=mesh,
        out_type=jax.ShapeDtypeStruct((3, N, C), jnp.float32),
        scratch_types=[
            pltpu.VMEM((2, L), jnp.int32),
            pltpu.VMEM((2, L), jnp.int32),
            pltpu.VMEM((2, L), jnp.float32),
            pltpu.VMEM((2, L), jnp.float32),
            pltpu.VMEM((2, 4, L), jnp.float32),
            pltpu.VMEM((2, 4 * L), jnp.int32),
            pltpu.VMEM((2, L, C), jnp.float32),
            pltpu.VMEM((2, L, C), jnp.float32),
            pltpu.VMEM((2, 4 * L, C), jnp.float32),
            pltpu.VMEM((2, L, C), jnp.float32),
            pltpu.SemaphoreType.DMA,
            pltpu.SemaphoreType.DMA,
            pltpu.SemaphoreType.DMA,
            pltpu.SemaphoreType.DMA,
            pltpu.SemaphoreType.DMA,
            pltpu.SemaphoreType.DMA,
            pltpu.SemaphoreType.DMA,
        ],
    )
    return f(tab, mpt, dyt, dxt, batch_ids, flat_inds)
